# baseline (speedup 1.0000x reference)
import dataclasses
import functools

import jax
import jax.numpy as jnp
from jax import lax
from jax.experimental import pallas as pl
from jax.experimental.pallas import tpu as pltpu
from jax.experimental.pallas import tpu_sc as plsc

_B, _H, _W = 2, 512, 512
_N = 32768
_C = 64
_NOFF = 9
_PADROWS = 512
_NT = _N + _PADROWS
_NC, _NS = 2, 16
_NW = _NC * _NS
_SPW = _N // _NW
_HALO = 640
_WIN = _SPW + 2 * _HALO
_QPW = _SPW * _NOFF
_GCHUNK = 128
_RB = 512
_TW = 128


def _sc_params(tc_tiling=True):
    cp = pltpu.CompilerParams()
    if "needs_layout_passes" in pltpu.CompilerParams.__dataclass_fields__:
        cp = dataclasses.replace(cp, needs_layout_passes=False)
    if not tc_tiling:
        cp = dataclasses.replace(cp, use_tc_tiling_on_sc=False)
    return cp


def _vmesh():
    return plsc.VectorSubcoreMesh(
        core_axis_name="c", subcore_axis_name="s", num_cores=_NC,
        num_subcores=_NS)


def _build_nbr(idx_padded):
    @functools.partial(
        pl.kernel,
        out_type=jax.ShapeDtypeStruct((_N * _NOFF,), jnp.int32),
        mesh=_vmesh(),
        scratch_types=[
            pltpu.VMEM((_WIN,), jnp.int32),
            pltpu.VMEM((_QPW,), jnp.int32),
        ],
        compiler_params=_sc_params(),
    )
    def k(idx_hbm, nbr_hbm, win_v, out_v):
        wid = lax.axis_index("s") * _NC + lax.axis_index("c")
        base = wid * _SPW
        pltpu.sync_copy(idx_hbm.at[pl.ds(base, _WIN)], win_v)
        lane = lax.iota(jnp.int32, 16)

        @pl.loop(0, _SPW, step=16)
        def _(s):
            lid = s + lane
            flat = win_v[pl.ds(_HALO + s, 16)]
            gid = base + lid
            cx = flat & (_W - 1)
            cy = (flat >> 9) & (_H - 1)
            for koff in range(_NOFF):
                dy, dx = koff // 3 - 1, koff % 3 - 1
                pos_out = lid * _NOFF + koff
                if dy == 0 and dx == 0:
                    plsc.store_scatter(out_v, [pos_out], gid)
                    continue
                t = flat + (dy * _W + dx)
                valid = None
                if dx == -1:
                    valid = cx > 0
                elif dx == 1:
                    valid = cx < _W - 1
                if dy == -1:
                    vy = cy > 0
                    valid = vy if valid is None else valid & vy
                elif dy == 1:
                    vy = cy < _H - 1
                    valid = vy if valid is None else valid & vy
                lo = jnp.zeros((16,), jnp.int32)
                hi = jnp.full((16,), _WIN, jnp.int32)
                for _i in range(12):
                    mid = (lo + hi) >> 1
                    v = plsc.load_gather(win_v, [mid])
                    cond = v < t
                    lo = jnp.where(cond, mid + 1, lo)
                    hi = jnp.where(cond, hi, mid)
                posc = jnp.minimum(lo, _WIN - 1)
                v = plsc.load_gather(win_v, [posc])
                hit = valid & (v == t)
                nbr = jnp.where(hit, base - _HALO + lo, _N)
                plsc.store_scatter(out_v, [pos_out], nbr)

        pltpu.sync_copy(out_v, nbr_hbm.at[pl.ds(wid * _QPW, _QPW)])

    return k(idx_padded)


def _gather_rows(table, nbr2d):
    @functools.partial(
        pl.kernel,
        out_type=jax.ShapeDtypeStruct((_N * _NOFF, _TW), table.dtype),
        mesh=_vmesh(),
        compiler_params=_sc_params(),
    )
    def k(tab_hbm, nbr_hbm, g_hbm):
        def body(i_vmem, o_vmem):
            pltpu.sync_copy(tab_hbm.at[i_vmem.at[0]], o_vmem)

        pltpu.emit_pipeline(
            body,
            grid=(_N * _NOFF // _GCHUNK,),
            in_specs=[pl.BlockSpec((1, _GCHUNK), lambda i: (0, i))],
            out_specs=[pl.BlockSpec((_GCHUNK, _TW), lambda i: (i, 0))],
            core_axis_name=("c", "s"),
            dimension_semantics=(pltpu.PARALLEL,),
        )(nbr_hbm, g_hbm)

    return k(table, nbr2d)


def _conv_relu_table(g, ws, b):
    nb = _N // _RB

    def body(g_ref, w_ref, b_ref, o_ref):
        j = pl.program_id(0)

        @pl.when(j < nb)
        def _():
            acc = jnp.dot(g_ref[...], w_ref[...],
                          preferred_element_type=jnp.float32)
            res = jnp.maximum(acc + b_ref[...], 0.0)
            o_ref[...] = jnp.concatenate(
                [res, jnp.zeros((_RB, _TW - _C), jnp.float32)], axis=1)

        @pl.when(j >= nb)
        def _():
            o_ref[...] = jnp.zeros_like(o_ref)

    return pl.pallas_call(
        body,
        grid=(nb + _PADROWS // _RB,),
        in_specs=[
            pl.BlockSpec((_RB, _NOFF * _TW), lambda j: (lax.min(j, nb - 1), 0)),
            pl.BlockSpec((_NOFF * _TW, _C), lambda j: (0, 0)),
            pl.BlockSpec((1, _C), lambda j: (0, 0)),
        ],
        out_specs=pl.BlockSpec((_RB, _TW), lambda j: (j, 0)),
        out_shape=jax.ShapeDtypeStruct((_NT, _TW), jnp.float32),
    )(g, ws, b)


def _conv_relu_resid(g, x, ws, b, wd):
    nb = _N // _RB

    def body(g_ref, x_ref, w_ref, b_ref, wd_ref, o_ref):
        a = jnp.dot(g_ref[...], w_ref[...], preferred_element_type=jnp.float32)
        a = jnp.maximum(a + b_ref[...], 0.0)
        r = jnp.dot(x_ref[...], wd_ref[...],
                    preferred_element_type=jnp.float32)
        o_ref[...] = jnp.maximum(a + r, 0.0)

    return pl.pallas_call(
        body,
        grid=(nb,),
        in_specs=[
            pl.BlockSpec((_RB, _NOFF * _TW), lambda j: (j, 0)),
            pl.BlockSpec((_RB, _C), lambda j: (j, 0)),
            pl.BlockSpec((_NOFF * _TW, _C), lambda j: (0, 0)),
            pl.BlockSpec((1, _C), lambda j: (0, 0)),
            pl.BlockSpec((_C, _C), lambda j: (0, 0)),
        ],
        out_specs=pl.BlockSpec((_RB, _C), lambda j: (j, 0)),
        out_shape=jax.ShapeDtypeStruct((_N, _C), jnp.float32),
    )(g, x, ws, b, wd)


def _pad_w(w):
    w9 = w.reshape(_NOFF, _C, _C)
    return jnp.concatenate(
        [w9, jnp.zeros((_NOFF, _TW - _C, _C), w.dtype)],
        axis=1).reshape(_NOFF * _TW, _C)


def kernel(features, indices, W1, b1, W2, b2, Wd):
    idx = indices.astype(jnp.int32)
    idx_padded = jnp.concatenate([
        jnp.full((_HALO,), -1, jnp.int32),
        idx,
        jnp.full((_HALO,), jnp.int32(0x3FFFFFFF)),
    ])
    nbr = _build_nbr(idx_padded).reshape(1, _N * _NOFF)

    x_ext = jnp.pad(features, ((0, _PADROWS), (0, _TW - _C)))
    w1s = _pad_w(W1)
    w2s = _pad_w(W2)

    g1 = _gather_rows(x_ext, nbr).reshape(_N, _NOFF * _TW)
    out1t = _conv_relu_table(g1, w1s, b1.reshape(1, _C))
    g2 = _gather_rows(out1t, nbr).reshape(_N, _NOFF * _TW)
    return _conv_relu_resid(g2, features, w2s, b2.reshape(1, _C), Wd)


# speedup vs baseline: 1.9732x; 1.9732x over previous
import dataclasses
import functools

import jax
import jax.numpy as jnp
from jax import lax
from jax.experimental import pallas as pl
from jax.experimental.pallas import tpu as pltpu
from jax.experimental.pallas import tpu_sc as plsc

_B, _H, _W = 2, 512, 512
_N = 32768
_C = 64
_NOFF = 9
_PADROWS = 512
_NT = _N + _PADROWS
_NC, _NS = 2, 16
_NW = _NC * _NS
_SPW = _N // _NW
_HALO = 640
_WIN = _SPW + 2 * _HALO
_QPW = _SPW * _NOFF
_GCHUNK = 128
_RB = 512
_TW = 64


def _sc_params(tc_tiling=True):
    cp = pltpu.CompilerParams()
    if "needs_layout_passes" in pltpu.CompilerParams.__dataclass_fields__:
        cp = dataclasses.replace(cp, needs_layout_passes=False)
    if not tc_tiling:
        cp = dataclasses.replace(cp, use_tc_tiling_on_sc=False)
    return cp


def _vmesh():
    return plsc.VectorSubcoreMesh(
        core_axis_name="c", subcore_axis_name="s", num_cores=_NC,
        num_subcores=_NS)


def _build_nbr(idx_padded):
    @functools.partial(
        pl.kernel,
        out_type=jax.ShapeDtypeStruct((_N * _NOFF,), jnp.int32),
        mesh=_vmesh(),
        scratch_types=[
            pltpu.VMEM((_WIN,), jnp.int32),
            pltpu.VMEM((_QPW,), jnp.int32),
        ],
        compiler_params=_sc_params(),
    )
    def k(idx_hbm, nbr_hbm, win_v, out_v):
        wid = lax.axis_index("s") * _NC + lax.axis_index("c")
        base = wid * _SPW
        pltpu.sync_copy(idx_hbm.at[pl.ds(base, _WIN)], win_v)
        lane = lax.iota(jnp.int32, 16)

        @pl.loop(0, _SPW, step=16)
        def _(s):
            lid = s + lane
            flat = win_v[pl.ds(_HALO + s, 16)]
            gid = base + lid
            cx = flat & (_W - 1)
            cy = (flat >> 9) & (_H - 1)
            for koff in range(_NOFF):
                dy, dx = koff // 3 - 1, koff % 3 - 1
                pos_out = lid * _NOFF + koff
                if dy == 0 and dx == 0:
                    plsc.store_scatter(out_v, [pos_out], gid)
                    continue
                t = flat + (dy * _W + dx)
                valid = None
                if dx == -1:
                    valid = cx > 0
                elif dx == 1:
                    valid = cx < _W - 1
                if dy == -1:
                    vy = cy > 0
                    valid = vy if valid is None else valid & vy
                elif dy == 1:
                    vy = cy < _H - 1
                    valid = vy if valid is None else valid & vy
                lo = jnp.zeros((16,), jnp.int32)
                hi = jnp.full((16,), _WIN, jnp.int32)
                for _i in range(12):
                    mid = (lo + hi) >> 1
                    v = plsc.load_gather(win_v, [mid])
                    cond = v < t
                    lo = jnp.where(cond, mid + 1, lo)
                    hi = jnp.where(cond, hi, mid)
                posc = jnp.minimum(lo, _WIN - 1)
                v = plsc.load_gather(win_v, [posc])
                hit = valid & (v == t)
                nbr = jnp.where(hit, base - _HALO + lo, _N)
                plsc.store_scatter(out_v, [pos_out], nbr)

        pltpu.sync_copy(out_v, nbr_hbm.at[pl.ds(wid * _QPW, _QPW)])

    return k(idx_padded)


def _gather_rows(table, nbr2d):
    wq = _QPW // _GCHUNK

    @functools.partial(
        pl.kernel,
        out_type=jax.ShapeDtypeStruct((_N * _NOFF, _TW), table.dtype),
        mesh=_vmesh(),
        compiler_params=_sc_params(tc_tiling=(_TW == 128)),
    )
    def k(tab_hbm, nbr_hbm, g_hbm):
        wid = lax.axis_index("s") * _NC + lax.axis_index("c")
        wbase = wid * wq

        def body(i_vmem, o_vmem):
            pltpu.sync_copy(tab_hbm.at[i_vmem.at[0]], o_vmem)

        pltpu.emit_pipeline(
            body,
            grid=(wq,),
            in_specs=[pl.BlockSpec((1, _GCHUNK), lambda i: (0, wbase + i))],
            out_specs=[pl.BlockSpec((_GCHUNK, _TW), lambda i: (wbase + i, 0))],
            dimension_semantics=(pltpu.ARBITRARY,),
        )(nbr_hbm, g_hbm)

    return k(table, nbr2d)


def _conv_relu_table(g, ws, b):
    nb = _N // _RB

    def body(g_ref, w_ref, b_ref, o_ref):
        j = pl.program_id(0)

        @pl.when(j < nb)
        def _():
            acc = jnp.dot(g_ref[...], w_ref[...],
                          preferred_element_type=jnp.float32)
            res = jnp.maximum(acc + b_ref[...], 0.0)
            if _TW > _C:
                res = jnp.concatenate(
                    [res, jnp.zeros((_RB, _TW - _C), jnp.float32)], axis=1)
            o_ref[...] = res

        @pl.when(j >= nb)
        def _():
            o_ref[...] = jnp.zeros_like(o_ref)

    return pl.pallas_call(
        body,
        grid=(nb + _PADROWS // _RB,),
        in_specs=[
            pl.BlockSpec((_RB, _NOFF * _TW), lambda j: (lax.min(j, nb - 1), 0)),
            pl.BlockSpec((_NOFF * _TW, _C), lambda j: (0, 0)),
            pl.BlockSpec((1, _C), lambda j: (0, 0)),
        ],
        out_specs=pl.BlockSpec((_RB, _TW), lambda j: (j, 0)),
        out_shape=jax.ShapeDtypeStruct((_NT, _TW), jnp.float32),
    )(g, ws, b)


def _conv_relu_resid(g, x, ws, b, wd):
    nb = _N // _RB

    def body(g_ref, x_ref, w_ref, b_ref, wd_ref, o_ref):
        a = jnp.dot(g_ref[...], w_ref[...], preferred_element_type=jnp.float32)
        a = jnp.maximum(a + b_ref[...], 0.0)
        r = jnp.dot(x_ref[...], wd_ref[...],
                    preferred_element_type=jnp.float32)
        o_ref[...] = jnp.maximum(a + r, 0.0)

    return pl.pallas_call(
        body,
        grid=(nb,),
        in_specs=[
            pl.BlockSpec((_RB, _NOFF * _TW), lambda j: (j, 0)),
            pl.BlockSpec((_RB, _C), lambda j: (j, 0)),
            pl.BlockSpec((_NOFF * _TW, _C), lambda j: (0, 0)),
            pl.BlockSpec((1, _C), lambda j: (0, 0)),
            pl.BlockSpec((_C, _C), lambda j: (0, 0)),
        ],
        out_specs=pl.BlockSpec((_RB, _C), lambda j: (j, 0)),
        out_shape=jax.ShapeDtypeStruct((_N, _C), jnp.float32),
    )(g, x, ws, b, wd)


def _pad_w(w):
    w9 = w.reshape(_NOFF, _C, _C)
    if _TW > _C:
        w9 = jnp.concatenate(
            [w9, jnp.zeros((_NOFF, _TW - _C, _C), w.dtype)], axis=1)
    return w9.reshape(_NOFF * _TW, _C)


def kernel(features, indices, W1, b1, W2, b2, Wd):
    idx = indices.astype(jnp.int32)
    idx_padded = jnp.concatenate([
        jnp.full((_HALO,), -1, jnp.int32),
        idx,
        jnp.full((_HALO,), jnp.int32(0x3FFFFFFF)),
    ])
    nbr = _build_nbr(idx_padded).reshape(1, _N * _NOFF)

    x_ext = jnp.pad(features, ((0, _PADROWS), (0, _TW - _C)))
    w1s = _pad_w(W1)
    w2s = _pad_w(W2)

    g1 = _gather_rows(x_ext, nbr).reshape(_N, _NOFF * _TW)
    out1t = _conv_relu_table(g1, w1s, b1.reshape(1, _C))
    g2 = _gather_rows(out1t, nbr).reshape(_N, _NOFF * _TW)
    return _conv_relu_resid(g2, features, w2s, b2.reshape(1, _C), Wd)


# speedup vs baseline: 1.9789x; 1.0029x over previous
import dataclasses
import functools

import jax
import jax.numpy as jnp
from jax import lax
from jax.experimental import pallas as pl
from jax.experimental.pallas import tpu as pltpu
from jax.experimental.pallas import tpu_sc as plsc

_B, _H, _W = 2, 512, 512
_N = 32768
_C = 64
_NOFF = 9
_PADROWS = 512
_NT = _N + _PADROWS
_NC, _NS = 2, 16
_NW = _NC * _NS
_SPW = _N // _NW
_HALO = 640
_WIN = _SPW + 2 * _HALO
_QPW = _SPW * _NOFF
_GCHUNK = 768
_RB = 512
_TW = 64


def _sc_params(tc_tiling=True):
    cp = pltpu.CompilerParams()
    if "needs_layout_passes" in pltpu.CompilerParams.__dataclass_fields__:
        cp = dataclasses.replace(cp, needs_layout_passes=False)
    if not tc_tiling:
        cp = dataclasses.replace(cp, use_tc_tiling_on_sc=False)
    return cp


def _vmesh():
    return plsc.VectorSubcoreMesh(
        core_axis_name="c", subcore_axis_name="s", num_cores=_NC,
        num_subcores=_NS)


def _build_nbr(idx_padded):
    @functools.partial(
        pl.kernel,
        out_type=jax.ShapeDtypeStruct((_N * _NOFF,), jnp.int32),
        mesh=_vmesh(),
        scratch_types=[
            pltpu.VMEM((_WIN,), jnp.int32),
            pltpu.VMEM((_QPW,), jnp.int32),
        ],
        compiler_params=_sc_params(),
    )
    def k(idx_hbm, nbr_hbm, win_v, out_v):
        wid = lax.axis_index("s") * _NC + lax.axis_index("c")
        base = wid * _SPW
        pltpu.sync_copy(idx_hbm.at[pl.ds(base, _WIN)], win_v)
        lane = lax.iota(jnp.int32, 16)

        @pl.loop(0, _SPW, step=16)
        def _(s):
            lid = s + lane
            flat = win_v[pl.ds(_HALO + s, 16)]
            gid = base + lid
            cx = flat & (_W - 1)
            cy = (flat >> 9) & (_H - 1)
            for koff in range(_NOFF):
                dy, dx = koff // 3 - 1, koff % 3 - 1
                pos_out = lid * _NOFF + koff
                if dy == 0 and dx == 0:
                    plsc.store_scatter(out_v, [pos_out], gid)
                    continue
                t = flat + (dy * _W + dx)
                valid = None
                if dx == -1:
                    valid = cx > 0
                elif dx == 1:
                    valid = cx < _W - 1
                if dy == -1:
                    vy = cy > 0
                    valid = vy if valid is None else valid & vy
                elif dy == 1:
                    vy = cy < _H - 1
                    valid = vy if valid is None else valid & vy
                lo = jnp.zeros((16,), jnp.int32)
                hi = jnp.full((16,), _WIN, jnp.int32)
                for _i in range(12):
                    mid = (lo + hi) >> 1
                    v = plsc.load_gather(win_v, [mid])
                    cond = v < t
                    lo = jnp.where(cond, mid + 1, lo)
                    hi = jnp.where(cond, hi, mid)
                posc = jnp.minimum(lo, _WIN - 1)
                v = plsc.load_gather(win_v, [posc])
                hit = valid & (v == t)
                nbr = jnp.where(hit, base - _HALO + lo, _N)
                plsc.store_scatter(out_v, [pos_out], nbr)

        pltpu.sync_copy(out_v, nbr_hbm.at[pl.ds(wid * _QPW, _QPW)])

    return k(idx_padded)


def _gather_rows(table, nbr2d):
    wq = _QPW // _GCHUNK

    @functools.partial(
        pl.kernel,
        out_type=jax.ShapeDtypeStruct((_N * _NOFF, _TW), table.dtype),
        mesh=_vmesh(),
        compiler_params=_sc_params(tc_tiling=(_TW == 128)),
    )
    def k(tab_hbm, nbr_hbm, g_hbm):
        wid = lax.axis_index("s") * _NC + lax.axis_index("c")
        wbase = wid * wq

        def body(i_vmem, o_vmem):
            pltpu.sync_copy(tab_hbm.at[i_vmem.at[0]], o_vmem)

        pltpu.emit_pipeline(
            body,
            grid=(wq,),
            in_specs=[pl.BlockSpec((1, _GCHUNK), lambda i: (0, wbase + i))],
            out_specs=[pl.BlockSpec((_GCHUNK, _TW), lambda i: (wbase + i, 0))],
            dimension_semantics=(pltpu.ARBITRARY,),
        )(nbr_hbm, g_hbm)

    return k(table, nbr2d)


def _conv_relu_table(g, ws, b):
    nb = _N // _RB

    def body(g_ref, w_ref, b_ref, o_ref):
        j = pl.program_id(0)

        @pl.when(j < nb)
        def _():
            acc = jnp.dot(g_ref[...], w_ref[...],
                          preferred_element_type=jnp.float32)
            res = jnp.maximum(acc + b_ref[...], 0.0)
            if _TW > _C:
                res = jnp.concatenate(
                    [res, jnp.zeros((_RB, _TW - _C), jnp.float32)], axis=1)
            o_ref[...] = res

        @pl.when(j >= nb)
        def _():
            o_ref[...] = jnp.zeros_like(o_ref)

    return pl.pallas_call(
        body,
        grid=(nb + _PADROWS // _RB,),
        in_specs=[
            pl.BlockSpec((_RB, _NOFF * _TW), lambda j: (lax.min(j, nb - 1), 0)),
            pl.BlockSpec((_NOFF * _TW, _C), lambda j: (0, 0)),
            pl.BlockSpec((1, _C), lambda j: (0, 0)),
        ],
        out_specs=pl.BlockSpec((_RB, _TW), lambda j: (j, 0)),
        out_shape=jax.ShapeDtypeStruct((_NT, _TW), jnp.float32),
    )(g, ws, b)


def _conv_relu_resid(g, x, ws, b, wd):
    nb = _N // _RB

    def body(g_ref, x_ref, w_ref, b_ref, wd_ref, o_ref):
        a = jnp.dot(g_ref[...], w_ref[...], preferred_element_type=jnp.float32)
        a = jnp.maximum(a + b_ref[...], 0.0)
        r = jnp.dot(x_ref[...], wd_ref[...],
                    preferred_element_type=jnp.float32)
        o_ref[...] = jnp.maximum(a + r, 0.0)

    return pl.pallas_call(
        body,
        grid=(nb,),
        in_specs=[
            pl.BlockSpec((_RB, _NOFF * _TW), lambda j: (j, 0)),
            pl.BlockSpec((_RB, _C), lambda j: (j, 0)),
            pl.BlockSpec((_NOFF * _TW, _C), lambda j: (0, 0)),
            pl.BlockSpec((1, _C), lambda j: (0, 0)),
            pl.BlockSpec((_C, _C), lambda j: (0, 0)),
        ],
        out_specs=pl.BlockSpec((_RB, _C), lambda j: (j, 0)),
        out_shape=jax.ShapeDtypeStruct((_N, _C), jnp.float32),
    )(g, x, ws, b, wd)


def _pad_w(w):
    w9 = w.reshape(_NOFF, _C, _C)
    if _TW > _C:
        w9 = jnp.concatenate(
            [w9, jnp.zeros((_NOFF, _TW - _C, _C), w.dtype)], axis=1)
    return w9.reshape(_NOFF * _TW, _C)


def kernel(features, indices, W1, b1, W2, b2, Wd):
    idx = indices.astype(jnp.int32)
    idx_padded = jnp.concatenate([
        jnp.full((_HALO,), -1, jnp.int32),
        idx,
        jnp.full((_HALO,), jnp.int32(0x3FFFFFFF)),
    ])
    nbr = _build_nbr(idx_padded).reshape(1, _N * _NOFF)

    x_ext = jnp.pad(features, ((0, _PADROWS), (0, _TW - _C)))
    w1s = _pad_w(W1)
    w2s = _pad_w(W2)

    g1 = _gather_rows(x_ext, nbr).reshape(_N, _NOFF * _TW)
    out1t = _conv_relu_table(g1, w1s, b1.reshape(1, _C))
    g2 = _gather_rows(out1t, nbr).reshape(_N, _NOFF * _TW)
    return _conv_relu_resid(g2, features, w2s, b2.reshape(1, _C), Wd)
